# baseline (speedup 1.0000x reference)
import functools

import jax
from jax import lax
from jax.experimental import pallas as pl
from jax.experimental.pallas import tpu as pltpu
from jax.experimental.pallas import tpu_sc as plsc

B_STATIC = 4
CHUNK = 64


def kernel(B, T, pe_weight):
    max_len, d_model = pe_weight.shape
    info = plsc.get_sparse_core_info()
    nc, ns = info.num_cores, info.num_subcores
    nw = nc * ns
    rows = max_len // nw
    nchunks = rows // CHUNK

    mesh = plsc.VectorSubcoreMesh(core_axis_name="c", subcore_axis_name="s")

    @functools.partial(
        pl.kernel,
        mesh=mesh,
        out_type=jax.ShapeDtypeStruct((B_STATIC, max_len, d_model), pe_weight.dtype),
        scratch_types=[
            pltpu.VMEM((CHUNK, d_model), pe_weight.dtype),
            pltpu.SemaphoreType.DMA,
        ],
    )
    def sc_copy(table_hbm, out_hbm, buf, sem):
        wid = lax.axis_index("s") * nc + lax.axis_index("c")
        base = wid * rows

        for i in range(nchunks):
            start = base + i * CHUNK
            pltpu.sync_copy(table_hbm.at[pl.ds(start, CHUNK)], buf)
            copies = [
                pltpu.async_copy(buf, out_hbm.at[b, pl.ds(start, CHUNK)], sem)
                for b in range(B_STATIC)
            ]
            for c in copies:
                c.wait()

    return sc_copy(pe_weight)


# speedup vs baseline: 1.0062x; 1.0062x over previous
import functools

import jax
from jax import lax
from jax.experimental import pallas as pl
from jax.experimental.pallas import tpu as pltpu
from jax.experimental.pallas import tpu_sc as plsc

B_STATIC = 4
CHUNK = 56


def kernel(B, T, pe_weight):
    max_len, d_model = pe_weight.shape
    info = plsc.get_sparse_core_info()
    nc, ns = info.num_cores, info.num_subcores
    nw = nc * ns
    rows = max_len // nw
    sizes = [CHUNK] * (rows // CHUNK)
    if rows % CHUNK:
        sizes.append(rows % CHUNK)
    offs = [sum(sizes[:i]) for i in range(len(sizes))]
    nchunks = len(sizes)

    mesh = plsc.VectorSubcoreMesh(core_axis_name="c", subcore_axis_name="s")

    @functools.partial(
        pl.kernel,
        mesh=mesh,
        out_type=jax.ShapeDtypeStruct((B_STATIC, max_len, d_model), pe_weight.dtype),
        scratch_types=[
            pltpu.VMEM((CHUNK, d_model), pe_weight.dtype),
            pltpu.VMEM((CHUNK, d_model), pe_weight.dtype),
            pltpu.SemaphoreType.DMA,
            pltpu.SemaphoreType.DMA,
            pltpu.SemaphoreType.DMA,
            pltpu.SemaphoreType.DMA,
        ],
    )
    def sc_copy(table_hbm, out_hbm, buf0, buf1, isem0, isem1, osem0, osem1):
        wid = lax.axis_index("s") * nc + lax.axis_index("c")
        base = wid * rows
        bufs = (buf0, buf1)
        isems = (isem0, isem1)
        osems = (osem0, osem1)

        def read(i):
            k = i % 2
            start = base + offs[i]
            return pltpu.async_copy(
                table_hbm.at[pl.ds(start, sizes[i])],
                bufs[k].at[pl.ds(0, sizes[i])],
                isems[k],
            )

        def write(i):
            k = i % 2
            start = base + offs[i]
            return [
                pltpu.async_copy(
                    bufs[k].at[pl.ds(0, sizes[i])],
                    out_hbm.at[b, pl.ds(start, sizes[i])],
                    osems[k],
                )
                for b in range(B_STATIC)
            ]

        rh = {0: read(0)}
        wh = {}
        for i in range(nchunks):
            if i + 1 < nchunks:
                if i >= 1:
                    for h in wh[i - 1]:
                        h.wait()
                rh[i + 1] = read(i + 1)
            rh[i].wait()
            wh[i] = write(i)
        for i in (nchunks - 2, nchunks - 1):
            for h in wh[i]:
                h.wait()

    return sc_copy(pe_weight)
